# baseline (speedup 1.0000x reference)
import functools

import jax
import jax.numpy as jnp
from jax import lax
from jax.experimental import pallas as pl
from jax.experimental.pallas import tpu as pltpu
from jax.experimental.pallas import tpu_sc as plsc

E = 8
K = 2
M = 128
BM = 256
NW = 32
CH = 32
NBUF = 3


def _gating_body(x_ref, wgt_ref, bg_ref, w_ref, i_ref, u_ref):
    logits = jnp.dot(x_ref[...], wgt_ref[...],
                     preferred_element_type=jnp.float32) + bg_ref[...]
    idx8 = lax.broadcasted_iota(jnp.int32, logits.shape, 1)
    m1 = jnp.max(logits, axis=1, keepdims=True)
    i1 = jnp.min(jnp.where(logits == m1, idx8, E), axis=1, keepdims=True)
    l2 = jnp.where(idx8 == i1, -1e30, logits)
    m2 = jnp.max(l2, axis=1, keepdims=True)
    i2 = jnp.min(jnp.where(l2 == m2, idx8, E), axis=1, keepdims=True)
    e2 = jnp.exp(m2 - m1)
    denom = 1.0 + e2
    w_ref[...] = jnp.concatenate([1.0 / denom, e2 / denom], axis=1)
    i_ref[...] = jnp.concatenate([i1, i2], axis=1)
    p = jnp.exp(logits - m1)
    p = p / jnp.sum(p, axis=1, keepdims=True)

    @pl.when(pl.program_id(0) == 0)
    def _():
        u_ref[...] = jnp.zeros_like(u_ref)

    u_ref[...] += jnp.sum(p, axis=0, keepdims=True)


def _gating(x_flat, wg_t, bg_row):
    t = x_flat.shape[0]
    d = x_flat.shape[1]
    grid = (t // BM,)
    return pl.pallas_call(
        _gating_body,
        grid=grid,
        in_specs=[
            pl.BlockSpec((BM, d), lambda r: (r, 0)),
            pl.BlockSpec((d, E), lambda r: (0, 0)),
            pl.BlockSpec((1, E), lambda r: (0, 0)),
        ],
        out_specs=[
            pl.BlockSpec((BM, K), lambda r: (r, 0)),
            pl.BlockSpec((BM, K), lambda r: (r, 0)),
            pl.BlockSpec((1, E), lambda r: (0, 0)),
        ],
        out_shape=[
            jax.ShapeDtypeStruct((t, K), jnp.float32),
            jax.ShapeDtypeStruct((t, K), jnp.int32),
            jax.ShapeDtypeStruct((1, E), jnp.float32),
        ],
    )(x_flat, wg_t, bg_row)


def _gather_rows(table, idx):
    n_rows = idx.shape[0]
    tail = table.shape[1:]
    per_w = n_rows // NW
    n_chunks = per_w // CH
    mesh = plsc.VectorSubcoreMesh(core_axis_name="c", subcore_axis_name="s")

    @functools.partial(
        pl.kernel,
        out_type=jax.ShapeDtypeStruct((n_rows,) + tail, table.dtype),
        mesh=mesh,
        scratch_types=[
            pltpu.VMEM((per_w,), jnp.int32),
        ] + [pltpu.VMEM((CH,) + tail, table.dtype) for _ in range(NBUF)]
          + [pltpu.SemaphoreType.DMA for _ in range(NBUF)],
    )
    def k(table_hbm, idx_hbm, out_hbm, idx_v, *bufsem):
        bufs = bufsem[:NBUF]
        sems = bufsem[NBUF:]
        wid = lax.axis_index("s") * 2 + lax.axis_index("c")
        base = wid * per_w
        pltpu.sync_copy(idx_hbm.at[pl.ds(base, per_w)], idx_v)
        copies = [None] * NBUF

        def start(c):
            s = c % NBUF
            copies[s] = pltpu.make_async_copy(
                table_hbm.at[idx_v.at[pl.ds(c * CH, CH)]], bufs[s], sems[s])
            copies[s].start()

        for c in range(min(NBUF, n_chunks)):
            start(c)
        for c in range(n_chunks):
            nxt = c + NBUF
            copies[c % NBUF].wait()
            pltpu.sync_copy(bufs[c % NBUF],
                            out_hbm.at[pl.ds(base + c * CH, CH)])
            if nxt < n_chunks:
                start(nxt)

    return k(table, idx)


def _dispatch_scatter(x_flat, pos_a3, pos_b3, cap):
    t, d = x_flat.shape
    tpw = t // NW
    n_chunks = tpw // CH
    mesh = plsc.VectorSubcoreMesh(core_axis_name="c", subcore_axis_name="s")

    @functools.partial(
        pl.kernel,
        out_type=jax.ShapeDtypeStruct((cap, d), jnp.float32),
        mesh=mesh,
        scratch_types=[
            pltpu.VMEM((n_chunks, CH), jnp.int32),
            pltpu.VMEM((n_chunks, CH), jnp.int32),
            pltpu.VMEM((CH, d), jnp.float32),
            pltpu.VMEM((CH, d), jnp.float32),
            pltpu.SemaphoreType.DMA,
            pltpu.SemaphoreType.DMA,
            pltpu.SemaphoreType.DMA,
            pltpu.SemaphoreType.DMA,
        ],
    )
    def k(x_hbm, pa_hbm, pb_hbm, out_hbm, ia_v, ib_v, buf0, buf1,
          sa0, sa1, sb0, sb1):
        wid = lax.axis_index("s") * 2 + lax.axis_index("c")
        tbase = wid * tpw
        pltpu.sync_copy(pa_hbm.at[wid], ia_v)
        pltpu.sync_copy(pb_hbm.at[wid], ib_v)
        bufs = (buf0, buf1)
        sas = (sa0, sa1)
        sbs = (sb0, sb1)
        copies = [None, None, None, None]
        for c in range(n_chunks):
            b = c % 2
            if c >= 2:
                copies[2 * b].wait()
                copies[2 * b + 1].wait()
            pltpu.sync_copy(x_hbm.at[pl.ds(tbase + c * CH, CH)], bufs[b])
            copies[2 * b] = pltpu.make_async_copy(
                bufs[b], out_hbm.at[ia_v.at[c]], sas[b])
            copies[2 * b].start()
            copies[2 * b + 1] = pltpu.make_async_copy(
                bufs[b], out_hbm.at[ib_v.at[c]], sbs[b])
            copies[2 * b + 1].start()
        for c in (n_chunks - 2, n_chunks - 1):
            b = c % 2
            copies[2 * b].wait()
            copies[2 * b + 1].wait()

    return k(x_flat, pos_a3, pos_b3)


def _cast_body(w_ref, o_ref):
    o_ref[...] = w_ref[...].astype(jnp.bfloat16)


def _cast_bf16(w, rows):
    e, a, b = w.shape
    return pl.pallas_call(
        _cast_body,
        grid=(e, a // rows),
        in_specs=[pl.BlockSpec((1, rows, b), lambda i, j: (i, j, 0))],
        out_specs=pl.BlockSpec((1, rows, b), lambda i, j: (i, j, 0)),
        out_shape=jax.ShapeDtypeStruct((e, a, b), jnp.bfloat16),
    )(w)


def _ffn_body(te_ref, xs_ref, w1_ref, b1_ref, w2_ref, b2_ref, ys_ref):
    xb = xs_ref[...].astype(jnp.bfloat16)
    h = jnp.dot(xb, w1_ref[0], preferred_element_type=jnp.float32) + b1_ref[0]
    h = jnp.maximum(h, 0.0).astype(jnp.bfloat16)
    y = jnp.dot(h, w2_ref[0], preferred_element_type=jnp.float32)
    ys_ref[...] = y + b2_ref[0]


def _ffn(xs, tile_expert, w1b, b1r, w2b, b2r):
    p, d = xs.shape
    f = w1b.shape[2]
    nt = p // M
    grid_spec = pltpu.PrefetchScalarGridSpec(
        num_scalar_prefetch=1,
        grid=(nt,),
        in_specs=[
            pl.BlockSpec((M, d), lambda r, te: (r, 0)),
            pl.BlockSpec((1, d, f), lambda r, te: (te[r], 0, 0)),
            pl.BlockSpec((1, 1, f), lambda r, te: (te[r], 0, 0)),
            pl.BlockSpec((1, f, d), lambda r, te: (te[r], 0, 0)),
            pl.BlockSpec((1, 1, d), lambda r, te: (te[r], 0, 0)),
        ],
        out_specs=pl.BlockSpec((M, d), lambda r, te: (r, 0)),
    )
    return pl.pallas_call(
        _ffn_body,
        grid_spec=grid_spec,
        out_shape=jax.ShapeDtypeStruct((p, d), jnp.float32),
    )(tile_expert, xs, w1b, b1r, w2b, b2r)


def _add_body(a_ref, b_ref, w_ref, o_ref):
    o_ref[...] = (a_ref[...] * w_ref[:, 0:1] + b_ref[...] * w_ref[:, 1:2])


def _pair_sum(s, wts, t, d):
    nb = t // 512
    return pl.pallas_call(
        _add_body,
        grid=(nb,),
        in_specs=[
            pl.BlockSpec((512, d), lambda r: (r, 0)),
            pl.BlockSpec((512, d), lambda r: (r + nb, 0)),
            pl.BlockSpec((512, K), lambda r: (r, 0)),
        ],
        out_specs=pl.BlockSpec((512, d), lambda r: (r, 0)),
        out_shape=jax.ShapeDtypeStruct((t, d), jnp.float32),
    )(s, s, wts)


def kernel(x, Wg, bg, W1, b1, W2, b2):
    bsz, seq, d = x.shape
    f = W1.shape[2]
    t = bsz * seq
    tk = t * K
    cap = tk + E * M

    x_flat = x.reshape(t, d)

    wts, idxs, usum = _gating(x_flat, Wg.T, bg.reshape(1, E))
    usage = usum[0] / t
    aux_loss = jnp.sum(usage ** 2) * E

    e_flat = idxs.reshape(-1)
    oh = (e_flat[:, None] == jnp.arange(E, dtype=jnp.int32)[None, :])
    csum = jnp.cumsum(oh.astype(jnp.int32), axis=0)
    counts = csum[-1]
    padded = ((counts + M - 1) // M) * M
    pstart = jnp.concatenate(
        [jnp.zeros(1, jnp.int32), jnp.cumsum(padded).astype(jnp.int32)])
    rank = jnp.take_along_axis(csum, e_flat[:, None], axis=1)[:, 0] - 1
    pos = pstart[e_flat] + rank
    pos_a = pos[0::K]
    pos_b = pos[1::K]
    tile_expert = jnp.searchsorted(
        pstart[1:], jnp.arange(cap // M, dtype=jnp.int32) * M,
        side="right").astype(jnp.int32)
    tile_expert = jnp.minimum(tile_expert, E - 1)

    nch = t // NW // CH
    xs = _dispatch_scatter(x_flat, pos_a.reshape(NW, nch, CH),
                           pos_b.reshape(NW, nch, CH), cap)

    w1b = _cast_bf16(W1, 1024)
    w2b = _cast_bf16(W2, 1024)
    ys = _ffn(xs, tile_expert, w1b, b1.reshape(E, 1, f),
              w2b, b2.reshape(E, 1, d))

    s = _gather_rows(ys, jnp.concatenate([pos_a, pos_b]))

    out = _pair_sum(s, wts, t, d)
    return out.reshape(bsz, seq, d), aux_loss


# speedup vs baseline: 1.0498x; 1.0498x over previous
import functools

import jax
import jax.numpy as jnp
from jax import lax
from jax.experimental import pallas as pl
from jax.experimental.pallas import tpu as pltpu
from jax.experimental.pallas import tpu_sc as plsc

E = 8
K = 2
M = 256
BM = 256
NW = 32
CH = 32
NBUF = 3


def _gating_body(x_ref, wgt_ref, bg_ref, w_ref, i_ref, r_ref, u_ref, c_ref):
    @pl.when(pl.program_id(0) == 0)
    def _():
        u_ref[...] = jnp.zeros_like(u_ref)
        c_ref[...] = jnp.zeros_like(c_ref)

    logits = jnp.dot(x_ref[...], wgt_ref[...],
                     preferred_element_type=jnp.float32) + bg_ref[...]
    idx8 = lax.broadcasted_iota(jnp.int32, logits.shape, 1)
    m1 = jnp.max(logits, axis=1, keepdims=True)
    i1 = jnp.min(jnp.where(logits == m1, idx8, E), axis=1, keepdims=True)
    l2 = jnp.where(idx8 == i1, -1e30, logits)
    m2 = jnp.max(l2, axis=1, keepdims=True)
    i2 = jnp.min(jnp.where(l2 == m2, idx8, E), axis=1, keepdims=True)
    e2 = jnp.exp(m2 - m1)
    denom = 1.0 + e2
    w_ref[...] = jnp.concatenate([1.0 / denom, e2 / denom], axis=1)
    i_ref[...] = jnp.concatenate([i1, i2], axis=1)
    oh1 = (idx8 == i1).astype(jnp.float32)
    oh2 = (idx8 == i2).astype(jnp.float32)
    ohc = oh1 + oh2
    row = lax.broadcasted_iota(jnp.int32, (BM, BM), 0)
    col = lax.broadcasted_iota(jnp.int32, (BM, BM), 1)
    tri = (col < row).astype(jnp.float32)
    pref = jnp.dot(tri, ohc, preferred_element_type=jnp.float32) + c_ref[...]
    rank1 = jnp.sum(pref * oh1, axis=1, keepdims=True)
    rank2 = jnp.sum(pref * oh2, axis=1, keepdims=True)
    r_ref[...] = jnp.concatenate([rank1, rank2], axis=1).astype(jnp.int32)
    c_ref[...] += jnp.sum(ohc, axis=0, keepdims=True)
    p = jnp.exp(logits - m1)
    p = p / jnp.sum(p, axis=1, keepdims=True)
    u_ref[...] += jnp.sum(p, axis=0, keepdims=True)


def _gating(x_flat, wg_t, bg_row):
    t = x_flat.shape[0]
    d = x_flat.shape[1]
    grid = (t // BM,)
    return pl.pallas_call(
        _gating_body,
        grid=grid,
        in_specs=[
            pl.BlockSpec((BM, d), lambda r: (r, 0)),
            pl.BlockSpec((d, E), lambda r: (0, 0)),
            pl.BlockSpec((1, E), lambda r: (0, 0)),
        ],
        out_specs=[
            pl.BlockSpec((BM, K), lambda r: (r, 0)),
            pl.BlockSpec((BM, K), lambda r: (r, 0)),
            pl.BlockSpec((BM, K), lambda r: (r, 0)),
            pl.BlockSpec((1, E), lambda r: (0, 0)),
            pl.BlockSpec((1, E), lambda r: (0, 0)),
        ],
        out_shape=[
            jax.ShapeDtypeStruct((t, K), jnp.float32),
            jax.ShapeDtypeStruct((t, K), jnp.int32),
            jax.ShapeDtypeStruct((t, K), jnp.int32),
            jax.ShapeDtypeStruct((1, E), jnp.float32),
            jax.ShapeDtypeStruct((1, E), jnp.float32),
        ],
    )(x_flat, wg_t, bg_row)


def _gather_rows(table, idx):
    n_rows = idx.shape[0]
    tail = table.shape[1:]
    per_w = n_rows // NW
    n_chunks = per_w // CH
    mesh = plsc.VectorSubcoreMesh(core_axis_name="c", subcore_axis_name="s")

    @functools.partial(
        pl.kernel,
        out_type=jax.ShapeDtypeStruct((n_rows,) + tail, table.dtype),
        mesh=mesh,
        scratch_types=[
            pltpu.VMEM((per_w,), jnp.int32),
        ] + [pltpu.VMEM((CH,) + tail, table.dtype) for _ in range(NBUF)]
          + [pltpu.SemaphoreType.DMA for _ in range(NBUF)],
    )
    def k(table_hbm, idx_hbm, out_hbm, idx_v, *bufsem):
        bufs = bufsem[:NBUF]
        sems = bufsem[NBUF:]
        wid = lax.axis_index("s") * 2 + lax.axis_index("c")
        base = wid * per_w
        pltpu.sync_copy(idx_hbm.at[pl.ds(base, per_w)], idx_v)
        copies = [None] * NBUF

        def start(c):
            s = c % NBUF
            copies[s] = pltpu.make_async_copy(
                table_hbm.at[idx_v.at[pl.ds(c * CH, CH)]], bufs[s], sems[s])
            copies[s].start()

        for c in range(min(NBUF, n_chunks)):
            start(c)
        for c in range(n_chunks):
            nxt = c + NBUF
            copies[c % NBUF].wait()
            pltpu.sync_copy(bufs[c % NBUF],
                            out_hbm.at[pl.ds(base + c * CH, CH)])
            if nxt < n_chunks:
                start(nxt)

    return k(table, idx)


def _dispatch_scatter(x_flat, pos_a3, pos_b3, cap):
    t, d = x_flat.shape
    tpw = t // NW
    n_chunks = tpw // CH
    mesh = plsc.VectorSubcoreMesh(core_axis_name="c", subcore_axis_name="s")

    @functools.partial(
        pl.kernel,
        out_type=jax.ShapeDtypeStruct((cap, d), jnp.float32),
        mesh=mesh,
        scratch_types=[
            pltpu.VMEM((n_chunks, CH), jnp.int32),
            pltpu.VMEM((n_chunks, CH), jnp.int32),
            pltpu.VMEM((CH, d), jnp.float32),
            pltpu.VMEM((CH, d), jnp.float32),
            pltpu.SemaphoreType.DMA,
            pltpu.SemaphoreType.DMA,
            pltpu.SemaphoreType.DMA,
            pltpu.SemaphoreType.DMA,
        ],
    )
    def k(x_hbm, pa_hbm, pb_hbm, out_hbm, ia_v, ib_v, buf0, buf1,
          sa0, sa1, sb0, sb1):
        wid = lax.axis_index("s") * 2 + lax.axis_index("c")
        tbase = wid * tpw
        pltpu.sync_copy(pa_hbm.at[wid], ia_v)
        pltpu.sync_copy(pb_hbm.at[wid], ib_v)
        bufs = (buf0, buf1)
        sas = (sa0, sa1)
        sbs = (sb0, sb1)
        copies = [None, None, None, None]
        for c in range(n_chunks):
            b = c % 2
            if c >= 2:
                copies[2 * b].wait()
                copies[2 * b + 1].wait()
            pltpu.sync_copy(x_hbm.at[pl.ds(tbase + c * CH, CH)], bufs[b])
            copies[2 * b] = pltpu.make_async_copy(
                bufs[b], out_hbm.at[ia_v.at[c]], sas[b])
            copies[2 * b].start()
            copies[2 * b + 1] = pltpu.make_async_copy(
                bufs[b], out_hbm.at[ib_v.at[c]], sbs[b])
            copies[2 * b + 1].start()
        for c in (n_chunks - 2, n_chunks - 1):
            b = c % 2
            copies[2 * b].wait()
            copies[2 * b + 1].wait()

    return k(x_flat, pos_a3, pos_b3)


def _cast_body(w_ref, o_ref):
    o_ref[...] = w_ref[...].astype(jnp.bfloat16)


def _cast_bf16(w, rows):
    e, a, b = w.shape
    return pl.pallas_call(
        _cast_body,
        grid=(e, a // rows),
        in_specs=[pl.BlockSpec((1, rows, b), lambda i, j: (i, j, 0))],
        out_specs=pl.BlockSpec((1, rows, b), lambda i, j: (i, j, 0)),
        out_shape=jax.ShapeDtypeStruct((e, a, b), jnp.bfloat16),
    )(w)


def _ffn_body(te_ref, xs_ref, w1_ref, b1_ref, w2_ref, b2_ref, ys_ref):
    xb = xs_ref[...].astype(jnp.bfloat16)
    h = jnp.dot(xb, w1_ref[0], preferred_element_type=jnp.float32) + b1_ref[0]
    h = jnp.maximum(h, 0.0).astype(jnp.bfloat16)
    y = jnp.dot(h, w2_ref[0], preferred_element_type=jnp.float32)
    ys_ref[...] = y + b2_ref[0]


def _ffn(xs, tile_expert, w1b, b1r, w2b, b2r):
    p, d = xs.shape
    f = w1b.shape[2]
    nt = p // M
    grid_spec = pltpu.PrefetchScalarGridSpec(
        num_scalar_prefetch=1,
        grid=(nt,),
        in_specs=[
            pl.BlockSpec((M, d), lambda r, te: (r, 0)),
            pl.BlockSpec((1, d, f), lambda r, te: (te[r], 0, 0)),
            pl.BlockSpec((1, 1, f), lambda r, te: (te[r], 0, 0)),
            pl.BlockSpec((1, f, d), lambda r, te: (te[r], 0, 0)),
            pl.BlockSpec((1, 1, d), lambda r, te: (te[r], 0, 0)),
        ],
        out_specs=pl.BlockSpec((M, d), lambda r, te: (r, 0)),
    )
    return pl.pallas_call(
        _ffn_body,
        grid_spec=grid_spec,
        out_shape=jax.ShapeDtypeStruct((p, d), jnp.float32),
    )(tile_expert, xs, w1b, b1r, w2b, b2r)


def _add_body(a_ref, b_ref, w_ref, o_ref):
    o_ref[...] = (a_ref[...] * w_ref[:, 0:1] + b_ref[...] * w_ref[:, 1:2])


def _pair_sum(s, wts, t, d):
    nb = t // 512
    return pl.pallas_call(
        _add_body,
        grid=(nb,),
        in_specs=[
            pl.BlockSpec((512, d), lambda r: (r, 0)),
            pl.BlockSpec((512, d), lambda r: (r + nb, 0)),
            pl.BlockSpec((512, K), lambda r: (r, 0)),
        ],
        out_specs=pl.BlockSpec((512, d), lambda r: (r, 0)),
        out_shape=jax.ShapeDtypeStruct((t, d), jnp.float32),
    )(s, s, wts)


def kernel(x, Wg, bg, W1, b1, W2, b2):
    bsz, seq, d = x.shape
    f = W1.shape[2]
    t = bsz * seq
    tk = t * K
    cap = tk + E * M

    x_flat = x.reshape(t, d)

    wts, idxs, ranks, usum, cnts = _gating(x_flat, Wg.T, bg.reshape(1, E))
    usage = usum[0] / t
    aux_loss = jnp.sum(usage ** 2) * E

    counts = cnts[0].astype(jnp.int32)
    padded = ((counts + M - 1) // M) * M
    pstart = jnp.concatenate(
        [jnp.zeros(1, jnp.int32), jnp.cumsum(padded).astype(jnp.int32)])
    pos_a = pstart[idxs[:, 0]] + ranks[:, 0]
    pos_b = pstart[idxs[:, 1]] + ranks[:, 1]
    tile_expert = jnp.searchsorted(
        pstart[1:], jnp.arange(cap // M, dtype=jnp.int32) * M,
        side="right").astype(jnp.int32)
    tile_expert = jnp.minimum(tile_expert, E - 1)

    nch = t // NW // CH
    xs = _dispatch_scatter(x_flat, pos_a.reshape(NW, nch, CH),
                           pos_b.reshape(NW, nch, CH), cap)

    w1b = _cast_bf16(W1, 1024)
    w2b = _cast_bf16(W2, 1024)
    ys = _ffn(xs, tile_expert, w1b, b1.reshape(E, 1, f),
              w2b, b2.reshape(E, 1, d))

    s = _gather_rows(ys, jnp.concatenate([pos_a, pos_b]))

    out = _pair_sum(s, wts, t, d)
    return out.reshape(bsz, seq, d), aux_loss


# speedup vs baseline: 1.0861x; 1.0346x over previous
import functools

import jax
import jax.numpy as jnp
from jax import lax
from jax.experimental import pallas as pl
from jax.experimental.pallas import tpu as pltpu
from jax.experimental.pallas import tpu_sc as plsc

E = 8
K = 2
M = 256
BM = 256
NW = 32
CH = 32
NBUF = 3


def _gating_body(x_ref, wgt_ref, bg_ref, w_ref, i_ref, r_ref, u_ref, c_ref):
    @pl.when(pl.program_id(0) == 0)
    def _():
        u_ref[...] = jnp.zeros_like(u_ref)
        c_ref[...] = jnp.zeros_like(c_ref)

    logits = jnp.dot(x_ref[...], wgt_ref[...],
                     preferred_element_type=jnp.float32) + bg_ref[...]
    idx8 = lax.broadcasted_iota(jnp.int32, logits.shape, 1)
    m1 = jnp.max(logits, axis=1, keepdims=True)
    i1 = jnp.min(jnp.where(logits == m1, idx8, E), axis=1, keepdims=True)
    l2 = jnp.where(idx8 == i1, -1e30, logits)
    m2 = jnp.max(l2, axis=1, keepdims=True)
    i2 = jnp.min(jnp.where(l2 == m2, idx8, E), axis=1, keepdims=True)
    e2 = jnp.exp(m2 - m1)
    denom = 1.0 + e2
    w_ref[...] = jnp.concatenate([1.0 / denom, e2 / denom], axis=1)
    i_ref[...] = jnp.concatenate([i1, i2], axis=1)
    oh1 = (idx8 == i1).astype(jnp.float32)
    oh2 = (idx8 == i2).astype(jnp.float32)
    ohc = oh1 + oh2
    row = lax.broadcasted_iota(jnp.int32, (BM, BM), 0)
    col = lax.broadcasted_iota(jnp.int32, (BM, BM), 1)
    tri = (col < row).astype(jnp.float32)
    pref = jnp.dot(tri, ohc, preferred_element_type=jnp.float32) + c_ref[...]
    rank1 = jnp.sum(pref * oh1, axis=1, keepdims=True)
    rank2 = jnp.sum(pref * oh2, axis=1, keepdims=True)
    r_ref[...] = jnp.concatenate([rank1, rank2], axis=1).astype(jnp.int32)
    c_ref[...] += jnp.sum(ohc, axis=0, keepdims=True)
    p = jnp.exp(logits - m1)
    p = p / jnp.sum(p, axis=1, keepdims=True)
    u_ref[...] += jnp.sum(p, axis=0, keepdims=True)


def _gating(x_flat, wg_t, bg_row):
    t = x_flat.shape[0]
    d = x_flat.shape[1]
    grid = (t // BM,)
    return pl.pallas_call(
        _gating_body,
        grid=grid,
        in_specs=[
            pl.BlockSpec((BM, d), lambda r: (r, 0)),
            pl.BlockSpec((d, E), lambda r: (0, 0)),
            pl.BlockSpec((1, E), lambda r: (0, 0)),
        ],
        out_specs=[
            pl.BlockSpec((BM, K), lambda r: (r, 0)),
            pl.BlockSpec((BM, K), lambda r: (r, 0)),
            pl.BlockSpec((BM, K), lambda r: (r, 0)),
            pl.BlockSpec((1, E), lambda r: (0, 0)),
            pl.BlockSpec((1, E), lambda r: (0, 0)),
        ],
        out_shape=[
            jax.ShapeDtypeStruct((t, K), jnp.float32),
            jax.ShapeDtypeStruct((t, K), jnp.int32),
            jax.ShapeDtypeStruct((t, K), jnp.int32),
            jax.ShapeDtypeStruct((1, E), jnp.float32),
            jax.ShapeDtypeStruct((1, E), jnp.float32),
        ],
    )(x_flat, wg_t, bg_row)


def _gather_rows(table, idx):
    n_rows = idx.shape[0]
    tail = table.shape[1:]
    per_w = n_rows // NW
    n_chunks = per_w // CH
    mesh = plsc.VectorSubcoreMesh(core_axis_name="c", subcore_axis_name="s")

    @functools.partial(
        pl.kernel,
        out_type=jax.ShapeDtypeStruct((n_rows,) + tail, table.dtype),
        mesh=mesh,
        scratch_types=[
            pltpu.VMEM((per_w,), jnp.int32),
        ] + [pltpu.VMEM((CH,) + tail, table.dtype) for _ in range(NBUF)]
          + [pltpu.SemaphoreType.DMA for _ in range(NBUF)],
    )
    def k(table_hbm, idx_hbm, out_hbm, idx_v, *bufsem):
        bufs = bufsem[:NBUF]
        sems = bufsem[NBUF:]
        wid = lax.axis_index("s") * 2 + lax.axis_index("c")
        base = wid * per_w
        pltpu.sync_copy(idx_hbm.at[pl.ds(base, per_w)], idx_v)
        copies = [None] * NBUF

        def start(c):
            s = c % NBUF
            copies[s] = pltpu.make_async_copy(
                table_hbm.at[idx_v.at[pl.ds(c * CH, CH)]], bufs[s], sems[s])
            copies[s].start()

        for c in range(min(NBUF, n_chunks)):
            start(c)
        for c in range(n_chunks):
            nxt = c + NBUF
            copies[c % NBUF].wait()
            pltpu.sync_copy(bufs[c % NBUF],
                            out_hbm.at[pl.ds(base + c * CH, CH)])
            if nxt < n_chunks:
                start(nxt)

    return k(table, idx)


def _dispatch_scatter(x_flat, pos_a3, pos_b3, cap):
    t, d = x_flat.shape
    tpw = t // NW
    n_chunks = tpw // CH
    mesh = plsc.VectorSubcoreMesh(core_axis_name="c", subcore_axis_name="s")

    @functools.partial(
        pl.kernel,
        out_type=jax.ShapeDtypeStruct((cap, d), jnp.float32),
        mesh=mesh,
        scratch_types=[
            pltpu.VMEM((n_chunks, CH), jnp.int32),
            pltpu.VMEM((n_chunks, CH), jnp.int32),
            pltpu.VMEM((CH, d), jnp.float32),
            pltpu.VMEM((CH, d), jnp.float32),
            pltpu.SemaphoreType.DMA,
            pltpu.SemaphoreType.DMA,
            pltpu.SemaphoreType.DMA,
            pltpu.SemaphoreType.DMA,
        ],
    )
    def k(x_hbm, pa_hbm, pb_hbm, out_hbm, ia_v, ib_v, buf0, buf1,
          sa0, sa1, sb0, sb1):
        wid = lax.axis_index("s") * 2 + lax.axis_index("c")
        tbase = wid * tpw
        pltpu.sync_copy(pa_hbm.at[wid], ia_v)
        pltpu.sync_copy(pb_hbm.at[wid], ib_v)
        bufs = (buf0, buf1)
        sas = (sa0, sa1)
        sbs = (sb0, sb1)
        copies = [None, None, None, None]
        for c in range(n_chunks):
            b = c % 2
            if c >= 2:
                copies[2 * b].wait()
                copies[2 * b + 1].wait()
            pltpu.sync_copy(x_hbm.at[pl.ds(tbase + c * CH, CH)], bufs[b])
            copies[2 * b] = pltpu.make_async_copy(
                bufs[b], out_hbm.at[ia_v.at[c]], sas[b])
            copies[2 * b].start()
            copies[2 * b + 1] = pltpu.make_async_copy(
                bufs[b], out_hbm.at[ib_v.at[c]], sbs[b])
            copies[2 * b + 1].start()
        for c in (n_chunks - 2, n_chunks - 1):
            b = c % 2
            copies[2 * b].wait()
            copies[2 * b + 1].wait()

    return k(x_flat, pos_a3, pos_b3)


def _cast_body(w_ref, dep_ref, o_ref):
    o_ref[...] = w_ref[...].astype(jnp.bfloat16)


def _cast_bf16(w, rows, dep):
    e, a, b = w.shape
    return pl.pallas_call(
        _cast_body,
        grid=(e, a // rows),
        in_specs=[pl.BlockSpec((1, rows, b), lambda i, j: (i, j, 0)),
                  pl.BlockSpec((1, E), lambda i, j: (0, 0))],
        out_specs=pl.BlockSpec((1, rows, b), lambda i, j: (i, j, 0)),
        out_shape=jax.ShapeDtypeStruct((e, a, b), jnp.bfloat16),
    )(w, dep)


def _ffn_body(te_ref, xs_ref, w1_ref, b1_ref, w2_ref, b2_ref, ys_ref):
    xb = xs_ref[...].astype(jnp.bfloat16)
    h = jnp.dot(xb, w1_ref[0], preferred_element_type=jnp.float32) + b1_ref[0]
    h = jnp.maximum(h, 0.0).astype(jnp.bfloat16)
    y = jnp.dot(h, w2_ref[0], preferred_element_type=jnp.float32)
    ys_ref[...] = y + b2_ref[0]


def _ffn(xs, tile_expert, w1b, b1r, w2b, b2r):
    p, d = xs.shape
    f = w1b.shape[2]
    nt = p // M
    grid_spec = pltpu.PrefetchScalarGridSpec(
        num_scalar_prefetch=1,
        grid=(nt,),
        in_specs=[
            pl.BlockSpec((M, d), lambda r, te: (r, 0)),
            pl.BlockSpec((1, d, f), lambda r, te: (te[r], 0, 0)),
            pl.BlockSpec((1, 1, f), lambda r, te: (te[r], 0, 0)),
            pl.BlockSpec((1, f, d), lambda r, te: (te[r], 0, 0)),
            pl.BlockSpec((1, 1, d), lambda r, te: (te[r], 0, 0)),
        ],
        out_specs=pl.BlockSpec((M, d), lambda r, te: (r, 0)),
    )
    return pl.pallas_call(
        _ffn_body,
        grid_spec=grid_spec,
        out_shape=jax.ShapeDtypeStruct((p, d), jnp.float32),
    )(tile_expert, xs, w1b, b1r, w2b, b2r)


def _add_body(a_ref, b_ref, w_ref, o_ref):
    o_ref[...] = (a_ref[...] * w_ref[:, 0:1] + b_ref[...] * w_ref[:, 1:2])


def _pair_sum(s, wts, t, d):
    nb = t // 512
    return pl.pallas_call(
        _add_body,
        grid=(nb,),
        in_specs=[
            pl.BlockSpec((512, d), lambda r: (r, 0)),
            pl.BlockSpec((512, d), lambda r: (r + nb, 0)),
            pl.BlockSpec((512, K), lambda r: (r, 0)),
        ],
        out_specs=pl.BlockSpec((512, d), lambda r: (r, 0)),
        out_shape=jax.ShapeDtypeStruct((t, d), jnp.float32),
    )(s, s, wts)


def kernel(x, Wg, bg, W1, b1, W2, b2):
    bsz, seq, d = x.shape
    f = W1.shape[2]
    t = bsz * seq
    tk = t * K
    cap = tk + E * M

    x_flat = x.reshape(t, d)

    wts, idxs, ranks, usum, cnts = _gating(x_flat, Wg.T, bg.reshape(1, E))
    usage = usum[0] / t
    aux_loss = jnp.sum(usage ** 2) * E

    counts = cnts[0].astype(jnp.int32)
    padded = ((counts + M - 1) // M) * M
    pstart = jnp.concatenate(
        [jnp.zeros(1, jnp.int32), jnp.cumsum(padded).astype(jnp.int32)])
    pos_a = pstart[idxs[:, 0]] + ranks[:, 0]
    pos_b = pstart[idxs[:, 1]] + ranks[:, 1]
    tile_starts = jnp.arange(cap // M, dtype=jnp.int32) * M
    tile_expert = jnp.sum(
        (tile_starts[:, None] >= pstart[None, 1:]).astype(jnp.int32), axis=1)
    tile_expert = jnp.minimum(tile_expert, E - 1)

    nch = t // NW // CH
    xs = _dispatch_scatter(x_flat, pos_a.reshape(NW, nch, CH),
                           pos_b.reshape(NW, nch, CH), cap)

    w1b = _cast_bf16(W1, 1024, cnts)
    w2b = _cast_bf16(W2, 1024, cnts)
    ys = _ffn(xs, tile_expert, w1b, b1.reshape(E, 1, f),
              w2b, b2.reshape(E, 1, d))

    s = _gather_rows(ys, jnp.concatenate([pos_a, pos_b]))

    out = _pair_sum(s, wts, t, d)
    return out.reshape(bsz, seq, d), aux_loss
